# baseline (speedup 1.0000x reference)
import jax
import jax.numpy as jnp
from jax import lax
from jax.experimental import pallas as pl
from jax.experimental.pallas import tpu as pltpu
from jax.experimental.pallas import tpu_sc as plsc

N = 100000
D = 16
HD = D // 2
TD = HD + 1
E = 3200000
CHUNK = 128
G = 5
GE = G * CHUNK
E_PAD = 3276800
NPAD = N + 352
GROUPS_PER_TILE = E_PAD // (16 * GE)
PAIRS_PER_TILE = GROUPS_PER_TILE // 2
ROWS_PER_TILE = NPAD // 16


def _mlp_body(x_ref, w1_ref, b1_ref, w2_ref, b2_ref, oa_ref, ob_ref):
    h = jnp.maximum(
        jnp.dot(x_ref[...], w1_ref[...], preferred_element_type=jnp.float32)
        + b1_ref[...], 0.0)
    y = (jnp.dot(h, w2_ref[...], preferred_element_type=jnp.float32)
         + b2_ref[...])
    ones = jnp.ones((y.shape[0], 1), jnp.float32)
    oa_ref[...] = jnp.concatenate([y[:, :HD], ones], axis=1)
    ob_ref[...] = jnp.concatenate([y[:, HD:], ones], axis=1)


def _mlp(x, w1, b1, w2, b2):
    blk = 10000
    grid = (N // blk,)
    return pl.pallas_call(
        _mlp_body,
        grid=grid,
        in_specs=[
            pl.BlockSpec((blk, D), lambda i: (i, 0)),
            pl.BlockSpec((D, D), lambda i: (0, 0)),
            pl.BlockSpec((1, D), lambda i: (0, 0)),
            pl.BlockSpec((D, D), lambda i: (0, 0)),
            pl.BlockSpec((1, D), lambda i: (0, 0)),
        ],
        out_specs=[
            pl.BlockSpec((blk, TD), lambda i: (i, 0)),
            pl.BlockSpec((blk, TD), lambda i: (i, 0)),
        ],
        out_shape=[
            jax.ShapeDtypeStruct((N, TD), jnp.float32),
            jax.ShapeDtypeStruct((N, TD), jnp.float32),
        ],
    )(x, w1, b1.reshape(1, D), w2, b2.reshape(1, D))


def _seg_body(table_hbm, srcg_hbm, dstg_hbm, z2_hbm,
              sums_hbm,
              src_v, dst_v, rows_v, acc_sh, isem, gsem, ssem0, ssem1):
    cid = lax.axis_index("c")
    sid = lax.axis_index("s")
    g0 = sid * GROUPS_PER_TILE
    tbl = table_hbm

    r0 = sid * ROWS_PER_TILE
    pltpu.sync_copy(z2_hbm.at[pl.ds(r0, ROWS_PER_TILE)],
                    acc_sh.at[pl.ds(r0, ROWS_PER_TILE)])
    plsc.subcore_barrier()

    del ssem1
    p0 = sid * PAIRS_PER_TILE

    def body(i, carry):
        p = p0 + i
        hi = [pltpu.async_copy(srcg_hbm.at[cid].at[p], src_v, isem),
              pltpu.async_copy(dstg_hbm.at[p], dst_v, isem)]
        for h in hi:
            h.wait()
        hg = [pltpu.async_copy(tbl.at[src_v.at[grp].at[b]],
                               rows_v.at[grp].at[pl.ds(b * CHUNK, CHUNK)],
                               gsem)
              for grp in range(2) for b in range(G)]
        for h in hg:
            h.wait()
        hs = [pltpu.async_copy(rows_v.at[grp].at[pl.ds(b * CHUNK, CHUNK)],
                               acc_sh.at[dst_v.at[grp].at[b]],
                               ssem0, add=True)
              for grp in range(2) for b in range(G)]
        for h in hs:
            h.wait()
        return carry

    lax.fori_loop(0, PAIRS_PER_TILE, body, 0)

    plsc.subcore_barrier()
    pltpu.sync_copy(acc_sh.at[pl.ds(r0, ROWS_PER_TILE)],
                    sums_hbm.at[cid].at[pl.ds(r0, ROWS_PER_TILE)])


def _segment(table2, srcg, dstg, z2):
    mesh = plsc.VectorSubcoreMesh(core_axis_name="c", subcore_axis_name="s")
    fn = pl.kernel(
        _seg_body, mesh=mesh,
        out_type=jax.ShapeDtypeStruct((2, NPAD, TD), jnp.float32),
        scratch_types=[
            pltpu.VMEM((2, G, CHUNK), jnp.int32),
            pltpu.VMEM((2, G, CHUNK), jnp.int32),
            pltpu.VMEM((2, GE, TD), jnp.float32),
            pltpu.VMEM_SHARED((NPAD, TD), jnp.float32),
            pltpu.SemaphoreType.DMA,
            pltpu.SemaphoreType.DMA,
            pltpu.SemaphoreType.DMA,
            pltpu.SemaphoreType.DMA,
        ],
        compiler_params=pltpu.CompilerParams(use_tc_tiling_on_sc=False),
    )
    return fn(table2, srcg, dstg, z2)


def _fin_body(p0_ref, p1_ref, a_ref, b_ref, wo_ref, bo_ref, o_ref):
    s0 = p0_ref[...] + a_ref[...]
    s1 = p1_ref[...] + b_ref[...]
    cnt = s0[:, HD:TD]
    mean = jnp.concatenate([s0[:, :HD], s1[:, :HD]], axis=1) / cnt
    o_ref[...] = (
        jnp.dot(mean, wo_ref[...], preferred_element_type=jnp.float32)
        + bo_ref[...])


def _finalize(p0, p1, ta, tb, wo, bo):
    blk = 2000
    grid = (N // blk,)
    return pl.pallas_call(
        _fin_body,
        grid=grid,
        in_specs=[
            pl.BlockSpec((blk, TD), lambda i: (i, 0)),
            pl.BlockSpec((blk, TD), lambda i: (i, 0)),
            pl.BlockSpec((blk, TD), lambda i: (i, 0)),
            pl.BlockSpec((blk, TD), lambda i: (i, 0)),
            pl.BlockSpec((D, D), lambda i: (0, 0)),
            pl.BlockSpec((1, D), lambda i: (0, 0)),
        ],
        out_specs=pl.BlockSpec((blk, D), lambda i: (i, 0)),
        out_shape=jax.ShapeDtypeStruct((N, D), jnp.float32),
    )(p0, p1, ta, tb, wo, bo.reshape(1, D))


@jax.jit
def kernel(edge_index, edge_attr,
           c1_W1, c1_b1, c1_W2, c1_b2, c1_Wo, c1_bo,
           c2_W1, c2_b1, c2_W2, c2_b2, c2_Wo, c2_bo):
    ta, tb = _mlp(edge_attr[:N], c2_W1, c2_b1, c2_W2, c2_b2)
    table2 = jnp.concatenate([ta, tb])

    pad = E_PAD - E
    src = jnp.concatenate([edge_index[0], jnp.zeros((pad,), jnp.int32)])
    dst = jnp.concatenate([edge_index[1], jnp.full((pad,), N, jnp.int32)])
    srcg = jnp.stack([src, src + N]).reshape(2, E_PAD // (2 * GE), 2, G, CHUNK)
    dstg = dst.reshape(E_PAD // (2 * GE), 2, G, CHUNK)
    z2 = jnp.zeros((NPAD, TD), jnp.float32)

    sums = _segment(table2, srcg, dstg, z2)

    return _finalize(sums[0, :N], sums[1, :N], ta, tb, c2_Wo, c2_bo)


# speedup vs baseline: 1.3457x; 1.3457x over previous
import jax
import jax.numpy as jnp
from jax import lax
from jax.experimental import pallas as pl
from jax.experimental.pallas import tpu as pltpu
from jax.experimental.pallas import tpu_sc as plsc

N = 100000
D = 16
HD = D // 2
TD = HD + 1
E = 3200000
CHUNK = 128
G = 8
GE = G * CHUNK
E_PAD = 3211264
NPAD = N + 352
BODIES_PER_TILE = E_PAD // (16 * GE)
ROWS_PER_TILE = NPAD // 16


def _mlp_body(x_ref, w1_ref, b1_ref, w2_ref, b2_ref, oa_ref, ob_ref):
    h = jnp.maximum(
        jnp.dot(x_ref[...], w1_ref[...], preferred_element_type=jnp.float32)
        + b1_ref[...], 0.0)
    y = (jnp.dot(h, w2_ref[...], preferred_element_type=jnp.float32)
         + b2_ref[...])
    ones = jnp.ones((y.shape[0], 1), jnp.float32)
    oa_ref[...] = jnp.concatenate([y[:, :HD], ones], axis=1)
    ob_ref[...] = jnp.concatenate([y[:, HD:], ones], axis=1)


def _mlp(x, w1, b1, w2, b2):
    blk = 10000
    grid = (N // blk,)
    return pl.pallas_call(
        _mlp_body,
        grid=grid,
        in_specs=[
            pl.BlockSpec((blk, D), lambda i: (i, 0)),
            pl.BlockSpec((D, D), lambda i: (0, 0)),
            pl.BlockSpec((1, D), lambda i: (0, 0)),
            pl.BlockSpec((D, D), lambda i: (0, 0)),
            pl.BlockSpec((1, D), lambda i: (0, 0)),
        ],
        out_specs=[
            pl.BlockSpec((blk, TD), lambda i: (i, 0)),
            pl.BlockSpec((blk, TD), lambda i: (i, 0)),
        ],
        out_shape=[
            jax.ShapeDtypeStruct((N, TD), jnp.float32),
            jax.ShapeDtypeStruct((N, TD), jnp.float32),
        ],
    )(x, w1, b1.reshape(1, D), w2, b2.reshape(1, D))


def _seg_body(table_hbm, srcg_hbm, dstg_hbm, z2_hbm,
              sums_hbm,
              src_v, dst_v, rows_v, acc_sh, isem, gsem, ssem0, ssem1):
    cid = lax.axis_index("c")
    sid = lax.axis_index("s")
    g0 = sid * BODIES_PER_TILE
    tbl = table_hbm

    r0 = sid * ROWS_PER_TILE
    pltpu.sync_copy(z2_hbm.at[pl.ds(r0, ROWS_PER_TILE)],
                    acc_sh.at[pl.ds(r0, ROWS_PER_TILE)])
    plsc.subcore_barrier()

    del ssem1
    b0 = sid * BODIES_PER_TILE

    def body(i, carry):
        p = b0 + i
        hi = [pltpu.async_copy(srcg_hbm.at[cid].at[p], src_v, isem),
              pltpu.async_copy(dstg_hbm.at[p], dst_v, isem)]
        for h in hi:
            h.wait()
        hg = [pltpu.async_copy(tbl.at[src_v.at[b]],
                               rows_v.at[pl.ds(b * CHUNK, CHUNK)], gsem)
              for b in range(G)]
        for h in hg:
            h.wait()
        hs = [pltpu.async_copy(rows_v.at[pl.ds(b * CHUNK, CHUNK)],
                               acc_sh.at[dst_v.at[b]],
                               ssem0, add=True)
              for b in range(G)]
        for h in hs:
            h.wait()
        return carry

    lax.fori_loop(0, BODIES_PER_TILE, body, 0)

    plsc.subcore_barrier()
    pltpu.sync_copy(acc_sh.at[pl.ds(r0, ROWS_PER_TILE)],
                    sums_hbm.at[cid].at[pl.ds(r0, ROWS_PER_TILE)])


def _segment(table2, srcg, dstg, z2):
    mesh = plsc.VectorSubcoreMesh(core_axis_name="c", subcore_axis_name="s")
    fn = pl.kernel(
        _seg_body, mesh=mesh,
        out_type=jax.ShapeDtypeStruct((2, NPAD, TD), jnp.float32),
        scratch_types=[
            pltpu.VMEM((G, CHUNK), jnp.int32),
            pltpu.VMEM((G, CHUNK), jnp.int32),
            pltpu.VMEM((GE, TD), jnp.float32),
            pltpu.VMEM_SHARED((NPAD, TD), jnp.float32),
            pltpu.SemaphoreType.DMA,
            pltpu.SemaphoreType.DMA,
            pltpu.SemaphoreType.DMA,
            pltpu.SemaphoreType.DMA,
        ],
        compiler_params=pltpu.CompilerParams(use_tc_tiling_on_sc=False),
    )
    return fn(table2, srcg, dstg, z2)


def _fin_body(p0_ref, p1_ref, a_ref, b_ref, wo_ref, bo_ref, o_ref):
    s0 = p0_ref[...] + a_ref[...]
    s1 = p1_ref[...] + b_ref[...]
    cnt = s0[:, HD:TD]
    mean = jnp.concatenate([s0[:, :HD], s1[:, :HD]], axis=1) / cnt
    o_ref[...] = (
        jnp.dot(mean, wo_ref[...], preferred_element_type=jnp.float32)
        + bo_ref[...])


def _finalize(p0, p1, ta, tb, wo, bo):
    blk = 2000
    grid = (N // blk,)
    return pl.pallas_call(
        _fin_body,
        grid=grid,
        in_specs=[
            pl.BlockSpec((blk, TD), lambda i: (i, 0)),
            pl.BlockSpec((blk, TD), lambda i: (i, 0)),
            pl.BlockSpec((blk, TD), lambda i: (i, 0)),
            pl.BlockSpec((blk, TD), lambda i: (i, 0)),
            pl.BlockSpec((D, D), lambda i: (0, 0)),
            pl.BlockSpec((1, D), lambda i: (0, 0)),
        ],
        out_specs=pl.BlockSpec((blk, D), lambda i: (i, 0)),
        out_shape=jax.ShapeDtypeStruct((N, D), jnp.float32),
    )(p0, p1, ta, tb, wo, bo.reshape(1, D))


@jax.jit
def kernel(edge_index, edge_attr,
           c1_W1, c1_b1, c1_W2, c1_b2, c1_Wo, c1_bo,
           c2_W1, c2_b1, c2_W2, c2_b2, c2_Wo, c2_bo):
    ta, tb = _mlp(edge_attr[:N], c2_W1, c2_b1, c2_W2, c2_b2)
    table2 = jnp.concatenate([ta, tb])

    pad = E_PAD - E
    src = jnp.concatenate([edge_index[0], jnp.zeros((pad,), jnp.int32)])
    dst = jnp.concatenate([edge_index[1], jnp.full((pad,), N, jnp.int32)])
    srcg = jnp.stack([src, src + N]).reshape(2, E_PAD // GE, G, CHUNK)
    dstg = dst.reshape(E_PAD // GE, G, CHUNK)
    z2 = jnp.zeros((NPAD, TD), jnp.float32)

    sums = _segment(table2, srcg, dstg, z2)

    return _finalize(sums[0, :N], sums[1, :N], ta, tb, c2_Wo, c2_bo)


# speedup vs baseline: 1.4997x; 1.1144x over previous
import jax
import jax.numpy as jnp
from jax import lax
from jax.experimental import pallas as pl
from jax.experimental.pallas import tpu as pltpu
from jax.experimental.pallas import tpu_sc as plsc

N = 100000
D = 16
HD = D // 2
TD = HD + 1
E = 3200000
CHUNK = 128
G = 8
GE = G * CHUNK
E_PAD = 3211264
NPAD = N + 352
BODIES_PER_TILE = E_PAD // (16 * GE)
ROWS_PER_TILE = NPAD // 16


def _mlp_body(x_ref, w1_ref, b1_ref, w2_ref, b2_ref, oa_ref, ob_ref):
    h = jnp.maximum(
        jnp.dot(x_ref[...], w1_ref[...], preferred_element_type=jnp.float32)
        + b1_ref[...], 0.0)
    y = (jnp.dot(h, w2_ref[...], preferred_element_type=jnp.float32)
         + b2_ref[...])
    ones = jnp.ones((y.shape[0], 1), jnp.float32)
    oa_ref[...] = jnp.concatenate([y[:, :HD], ones], axis=1)
    ob_ref[...] = jnp.concatenate([y[:, HD:], ones], axis=1)


def _mlp(x, w1, b1, w2, b2):
    blk = 10000
    grid = (N // blk,)
    return pl.pallas_call(
        _mlp_body,
        grid=grid,
        in_specs=[
            pl.BlockSpec((blk, D), lambda i: (i, 0)),
            pl.BlockSpec((D, D), lambda i: (0, 0)),
            pl.BlockSpec((1, D), lambda i: (0, 0)),
            pl.BlockSpec((D, D), lambda i: (0, 0)),
            pl.BlockSpec((1, D), lambda i: (0, 0)),
        ],
        out_specs=[
            pl.BlockSpec((blk, TD), lambda i: (i, 0)),
            pl.BlockSpec((blk, TD), lambda i: (i, 0)),
        ],
        out_shape=[
            jax.ShapeDtypeStruct((N, TD), jnp.float32),
            jax.ShapeDtypeStruct((N, TD), jnp.float32),
        ],
    )(x, w1, b1.reshape(1, D), w2, b2.reshape(1, D))


def _seg_body(table_hbm, srcg_hbm, dstg_hbm, z2_hbm,
              sums_hbm,
              src_v, dst_v, rows_v, acc_sh, isem, gsem, ssem0, ssem1):
    cid = lax.axis_index("c")
    sid = lax.axis_index("s")
    g0 = sid * BODIES_PER_TILE
    tbl = table_hbm

    r0 = sid * ROWS_PER_TILE
    for z in range(8):
        pltpu.sync_copy(z2_hbm,
                        acc_sh.at[pl.ds(r0 + z * (ROWS_PER_TILE // 8),
                                        ROWS_PER_TILE // 8)])
    plsc.subcore_barrier()

    del ssem1
    b0 = sid * BODIES_PER_TILE

    def fetch_idx(islot, p):
        pltpu.async_copy(srcg_hbm.at[cid].at[p], src_v.at[islot], isem)
        pltpu.async_copy(dstg_hbm.at[p], dst_v.at[islot], isem)

    def wait_idx(islot):
        pltpu.make_async_copy(srcg_hbm.at[0].at[0], src_v.at[islot],
                              isem).wait()
        pltpu.make_async_copy(dstg_hbm.at[0], dst_v.at[islot], isem).wait()

    def run(islot, prefetch):
        wait_idx(islot)
        hg = [pltpu.async_copy(tbl.at[src_v.at[islot].at[b]],
                               rows_v.at[pl.ds(b * CHUNK, CHUNK)], gsem)
              for b in range(G)]
        for h in hg:
            h.wait()
        if prefetch is not None:
            fetch_idx(1 - islot, prefetch)
        hs = [pltpu.async_copy(rows_v.at[pl.ds(b * CHUNK, CHUNK)],
                               acc_sh.at[dst_v.at[islot].at[b]],
                               ssem0, add=True)
              for b in range(G)]
        for h in hs:
            h.wait()

    fetch_idx(0, b0)

    def pair_body(i, carry):
        gbase = 2 * i
        run(0, b0 + gbase + 1)
        run(1, b0 + gbase + 2)
        return carry

    lax.fori_loop(0, BODIES_PER_TILE // 2 - 1, pair_body, 0)
    run(0, b0 + BODIES_PER_TILE - 1)
    run(1, None)

    plsc.subcore_barrier()
    pltpu.sync_copy(acc_sh.at[pl.ds(r0, ROWS_PER_TILE)],
                    sums_hbm.at[cid].at[pl.ds(r0, ROWS_PER_TILE)])


def _segment(table2, srcg, dstg, z2):
    mesh = plsc.VectorSubcoreMesh(core_axis_name="c", subcore_axis_name="s")
    fn = pl.kernel(
        _seg_body, mesh=mesh,
        out_type=jax.ShapeDtypeStruct((2, NPAD, TD), jnp.float32),
        scratch_types=[
            pltpu.VMEM((2, G, CHUNK), jnp.int32),
            pltpu.VMEM((2, G, CHUNK), jnp.int32),
            pltpu.VMEM((GE, TD), jnp.float32),
            pltpu.VMEM_SHARED((NPAD, TD), jnp.float32),
            pltpu.SemaphoreType.DMA,
            pltpu.SemaphoreType.DMA,
            pltpu.SemaphoreType.DMA,
            pltpu.SemaphoreType.DMA,
        ],
        compiler_params=pltpu.CompilerParams(use_tc_tiling_on_sc=False),
    )
    return fn(table2, srcg, dstg, z2)


def _fin_body(p0_ref, p1_ref, a_ref, b_ref, wo_ref, bo_ref, o_ref):
    s0 = p0_ref[...] + a_ref[...]
    s1 = p1_ref[...] + b_ref[...]
    cnt = s0[:, HD:TD]
    mean = jnp.concatenate([s0[:, :HD], s1[:, :HD]], axis=1) / cnt
    o_ref[...] = (
        jnp.dot(mean, wo_ref[...], preferred_element_type=jnp.float32)
        + bo_ref[...])


def _finalize(p0, p1, ta, tb, wo, bo):
    blk = 2000
    grid = (N // blk,)
    return pl.pallas_call(
        _fin_body,
        grid=grid,
        in_specs=[
            pl.BlockSpec((blk, TD), lambda i: (i, 0)),
            pl.BlockSpec((blk, TD), lambda i: (i, 0)),
            pl.BlockSpec((blk, TD), lambda i: (i, 0)),
            pl.BlockSpec((blk, TD), lambda i: (i, 0)),
            pl.BlockSpec((D, D), lambda i: (0, 0)),
            pl.BlockSpec((1, D), lambda i: (0, 0)),
        ],
        out_specs=pl.BlockSpec((blk, D), lambda i: (i, 0)),
        out_shape=jax.ShapeDtypeStruct((N, D), jnp.float32),
    )(p0, p1, ta, tb, wo, bo.reshape(1, D))


@jax.jit
def kernel(edge_index, edge_attr,
           c1_W1, c1_b1, c1_W2, c1_b2, c1_Wo, c1_bo,
           c2_W1, c2_b1, c2_W2, c2_b2, c2_Wo, c2_bo):
    ta, tb = _mlp(edge_attr[:N], c2_W1, c2_b1, c2_W2, c2_b2)
    table2 = jnp.concatenate([ta, tb])

    pad = E_PAD - E
    src = jnp.concatenate([edge_index[0], jnp.zeros((pad,), jnp.int32)])
    dst = jnp.concatenate([edge_index[1], jnp.full((pad,), N, jnp.int32)])
    srcg = jnp.stack([src, src + N]).reshape(2, E_PAD // GE, G, CHUNK)
    dstg = dst.reshape(E_PAD // GE, G, CHUNK)
    z2 = jnp.zeros((NPAD // 16 // 8, TD), jnp.float32)

    sums = _segment(table2, srcg, dstg, z2)

    return _finalize(sums[0, :N], sums[1, :N], ta, tb, c2_Wo, c2_bo)
